# baseline (speedup 1.0000x reference)
import jax
import jax.numpy as jnp
from jax.experimental import pallas as pl
from jax.experimental.pallas import tpu as pltpu


def _vae_body(inv_t_ref, x_ref, u_ref, w_ref, be_ref, wd_ref, bd_ref,
              px_ref, qz_ref):
    eps = 1e-20
    qz = x_ref[0] * w_ref[:] + be_ref[:]
    g = -jnp.log(-jnp.log(u_ref[0] + eps) + eps)
    z = (qz + g) * inv_t_ref[0, 0]
    z = z - jnp.max(z, axis=0, keepdims=True)
    e = jnp.exp(z)
    y = e / jnp.sum(e, axis=0, keepdims=True)
    px = jnp.dot(wd_ref[:], y, preferred_element_type=jnp.float32)
    px_ref[0] = px + bd_ref[:]
    qz_ref[0] = qz


def kernel(x, temperature, U, w_enc, b_enc, W_dec, b_dec):
    B, L = x.shape
    CAT = w_enc.shape[0]
    OUT = W_dec.shape[0]
    inv_t = (jnp.float32(1.0) / jnp.asarray(temperature, jnp.float32)).reshape(1, 1)
    w = w_enc.reshape(CAT, 1)
    be = b_enc.reshape(CAT, 1)
    bd = b_dec.reshape(OUT, 1)

    TL = 2048
    px, qz = pl.pallas_call(
        _vae_body,
        grid=(B, L // TL),
        in_specs=[
            pl.BlockSpec(memory_space=pltpu.SMEM),
            pl.BlockSpec((1, 1, TL), lambda b, l: (b, 0, l)),
            pl.BlockSpec((1, CAT, TL), lambda b, l: (b, 0, l)),
            pl.BlockSpec((CAT, 1), lambda b, l: (0, 0)),
            pl.BlockSpec((CAT, 1), lambda b, l: (0, 0)),
            pl.BlockSpec((OUT, CAT), lambda b, l: (0, 0)),
            pl.BlockSpec((OUT, 1), lambda b, l: (0, 0)),
        ],
        out_specs=[
            pl.BlockSpec((1, OUT, TL), lambda b, l: (b, 0, l)),
            pl.BlockSpec((1, CAT, TL), lambda b, l: (b, 0, l)),
        ],
        out_shape=[
            jax.ShapeDtypeStruct((B, OUT, L), jnp.float32),
            jax.ShapeDtypeStruct((B, CAT, L), jnp.float32),
        ],
        compiler_params=pltpu.CompilerParams(
            dimension_semantics=("parallel", "parallel"),
        ),
    )(inv_t, x.reshape(B, 1, L), U, w, be, W_dec, bd)
    return (px, qz)


# speedup vs baseline: 1.3626x; 1.3626x over previous
import jax
import jax.numpy as jnp
from jax.experimental import pallas as pl
from jax.experimental.pallas import tpu as pltpu

_TB = 2


def _vae_body(inv_t_ref, x_ref, u_ref, w_ref, be_ref, wd_ref, bd_ref,
              px_ref, qz_ref):
    eps = 1e-20
    for i in range(_TB):
        qz = x_ref[i] * w_ref[:] + be_ref[:]
        g = -jnp.log(-jnp.log(u_ref[i] + eps) + eps)
        z = (qz + g) * inv_t_ref[0, 0]
        z = z - jnp.max(z, axis=0, keepdims=True)
        e = jnp.exp(z)
        y = e / jnp.sum(e, axis=0, keepdims=True)
        px = jnp.dot(wd_ref[:], y, preferred_element_type=jnp.float32)
        px_ref[i] = px + bd_ref[:]
        qz_ref[i] = qz


def kernel(x, temperature, U, w_enc, b_enc, W_dec, b_dec):
    B, L = x.shape
    CAT = w_enc.shape[0]
    OUT = W_dec.shape[0]
    inv_t = (jnp.float32(1.0) / jnp.asarray(temperature, jnp.float32)).reshape(1, 1)
    w = w_enc.reshape(CAT, 1)
    be = b_enc.reshape(CAT, 1)
    bd = b_dec.reshape(OUT, 1)

    px, qz = pl.pallas_call(
        _vae_body,
        grid=(B // _TB,),
        in_specs=[
            pl.BlockSpec(memory_space=pltpu.SMEM),
            pl.BlockSpec((_TB, 1, L), lambda b: (b, 0, 0)),
            pl.BlockSpec((_TB, CAT, L), lambda b: (b, 0, 0)),
            pl.BlockSpec((CAT, 1), lambda b: (0, 0)),
            pl.BlockSpec((CAT, 1), lambda b: (0, 0)),
            pl.BlockSpec((OUT, CAT), lambda b: (0, 0)),
            pl.BlockSpec((OUT, 1), lambda b: (0, 0)),
        ],
        out_specs=[
            pl.BlockSpec((_TB, OUT, L), lambda b: (b, 0, 0)),
            pl.BlockSpec((_TB, CAT, L), lambda b: (b, 0, 0)),
        ],
        out_shape=[
            jax.ShapeDtypeStruct((B, OUT, L), jnp.float32),
            jax.ShapeDtypeStruct((B, CAT, L), jnp.float32),
        ],
        compiler_params=pltpu.CompilerParams(
            dimension_semantics=("parallel",),
        ),
    )(inv_t, x.reshape(B, 1, L), U, w, be, W_dec, bd)
    return (px, qz)


# speedup vs baseline: 1.3803x; 1.0130x over previous
import jax
import jax.numpy as jnp
from jax.experimental import pallas as pl
from jax.experimental.pallas import tpu as pltpu

_TB = 4


def _vae_body(inv_t_ref, x_ref, u_ref, w_ref, be_ref, wd_ref, bd_ref,
              px_ref, qz_ref):
    eps = 1e-20
    for i in range(_TB):
        qz = x_ref[i] * w_ref[:] + be_ref[:]
        g = -jnp.log(-jnp.log(u_ref[i] + eps) + eps)
        z = (qz + g) * inv_t_ref[0, 0]
        z = z - jnp.max(z, axis=0, keepdims=True)
        e = jnp.exp(z)
        y = e / jnp.sum(e, axis=0, keepdims=True)
        px = jnp.dot(wd_ref[:], y, preferred_element_type=jnp.float32)
        px_ref[i] = px + bd_ref[:]
        qz_ref[i] = qz


def kernel(x, temperature, U, w_enc, b_enc, W_dec, b_dec):
    B, L = x.shape
    CAT = w_enc.shape[0]
    OUT = W_dec.shape[0]
    inv_t = (jnp.float32(1.0) / jnp.asarray(temperature, jnp.float32)).reshape(1, 1)
    w = w_enc.reshape(CAT, 1)
    be = b_enc.reshape(CAT, 1)
    bd = b_dec.reshape(OUT, 1)

    px, qz = pl.pallas_call(
        _vae_body,
        grid=(B // _TB,),
        in_specs=[
            pl.BlockSpec(memory_space=pltpu.SMEM),
            pl.BlockSpec((_TB, 1, L), lambda b: (b, 0, 0)),
            pl.BlockSpec((_TB, CAT, L), lambda b: (b, 0, 0)),
            pl.BlockSpec((CAT, 1), lambda b: (0, 0)),
            pl.BlockSpec((CAT, 1), lambda b: (0, 0)),
            pl.BlockSpec((OUT, CAT), lambda b: (0, 0)),
            pl.BlockSpec((OUT, 1), lambda b: (0, 0)),
        ],
        out_specs=[
            pl.BlockSpec((_TB, OUT, L), lambda b: (b, 0, 0)),
            pl.BlockSpec((_TB, CAT, L), lambda b: (b, 0, 0)),
        ],
        out_shape=[
            jax.ShapeDtypeStruct((B, OUT, L), jnp.float32),
            jax.ShapeDtypeStruct((B, CAT, L), jnp.float32),
        ],
        compiler_params=pltpu.CompilerParams(
            dimension_semantics=("parallel",),
        ),
    )(inv_t, x.reshape(B, 1, L), U, w, be, W_dec, bd)
    return (px, qz)
